# baseline (speedup 1.0000x reference)
import jax
import jax.numpy as jnp
from jax import lax
from jax.experimental import pallas as pl
from jax.experimental.pallas import tpu as pltpu
from jax.experimental.pallas import tpu_sc as plsc

NC = 2
NS = 16
L = 16
CH = 80
DH = 64
CAP = 1040
RPT = 625
_R = 1000

_SC_PARAMS = pltpu.CompilerParams(
    needs_layout_passes=False, use_tc_tiling_on_sc=False)


def _sc_prep(src, dst, n, e):
  nw = NC * NS
  epw = e // nw
  nb = n // _R
  mesh = plsc.VectorSubcoreMesh(core_axis_name="c", subcore_axis_name="s")

  def body(src_h, dst_h, degp_o, bls_o, bld_o, cnt_o,
           sidxv, didx, deg, lsrc, ldst, cntv):
    c = lax.axis_index("c")
    s = lax.axis_index("s")
    wid = s * NC + c
    pltpu.sync_copy(src_h.at[pl.ds(wid * epw, epw)], sidxv)
    pltpu.sync_copy(dst_h.at[pl.ds(wid * epw, epw)], didx)

    def pb(i, carry):
      v = sidxv[pl.ds(i * L, L)]
      sidxv[pl.ds(i * L, L)] = (v & 15) * RPT + (v >> 4)
      return carry
    lax.fori_loop(0, epw // L, pb, 0)

    zeros = jnp.zeros((L,), jnp.float32)

    def zb(i, carry):
      deg[pl.ds(i * L, L)] = zeros
      return carry
    lax.fori_loop(0, n // L, zb, 0)

    ones = jnp.ones((L,), jnp.float32)

    def hb(i, carry):
      d = didx[pl.ds(i * L, L)]
      pd = (d & 15) * RPT + (d >> 4)
      plsc.addupdate_scatter(deg, [pd], ones)
      return carry
    lax.fori_loop(0, epw // L, hb, 0)
    for b in range(nb):
      pltpu.sync_copy(deg.at[pl.ds(b * _R, _R)], degp_o.at[b].at[wid])

    padd = jnp.full((L,), RPT, jnp.int32)
    zi = jnp.zeros((L,), jnp.int32)

    def fb(i, carry):
      lsrc[pl.ds(i * L, L)] = zi
      ldst[pl.ds(i * L, L)] = padd
      return carry
    lax.fori_loop(0, 16 * CAP // L, fb, 0)

    iot = lax.iota(jnp.int32, 16)
    cvec = jnp.zeros((16,), jnp.int32)
    cap_end = CAP - L
    for t in range(16):
      def bp(i, off, _t=t):
        d = didx[pl.ds(i * L, L)]
        m = (d & 15) == _t
        offc = jnp.minimum(off, _t * CAP + cap_end)
        plsc.store_compressed(lsrc.at[pl.ds(offc, L)],
                              sidxv[pl.ds(i * L, L)], mask=m)
        plsc.store_compressed(ldst.at[pl.ds(offc, L)], d >> 4, mask=m)
        return off + jnp.sum(m.astype(jnp.int32))
      off_end = lax.fori_loop(0, epw // L, bp, jnp.int32(t * CAP))
      offc2 = jnp.minimum(off_end, t * CAP + cap_end)
      lsrc[pl.ds(offc2, L)] = zi
      ldst[pl.ds(offc2, L)] = padd
      cvec = jnp.where(iot == t, off_end - t * CAP, cvec)
    cntv[...] = cvec
    pltpu.sync_copy(cntv, cnt_o.at[wid])
    pltpu.sync_copy(lsrc, bls_o.at[wid])
    pltpu.sync_copy(ldst, bld_o.at[wid])

  f = pl.kernel(
      body,
      out_type=[
          jax.ShapeDtypeStruct((nb, nw, _R), jnp.float32),
          jax.ShapeDtypeStruct((nw, 16 * CAP), jnp.int32),
          jax.ShapeDtypeStruct((nw, 16 * CAP), jnp.int32),
          jax.ShapeDtypeStruct((nw, 16), jnp.int32),
      ],
      mesh=mesh,
      compiler_params=_SC_PARAMS,
      scratch_types=[
          pltpu.VMEM((epw,), jnp.int32),
          pltpu.VMEM((epw,), jnp.int32),
          pltpu.VMEM((n,), jnp.float32),
          pltpu.VMEM((16 * CAP,), jnp.int32),
          pltpu.VMEM((16 * CAP,), jnp.int32),
          pltpu.VMEM((16,), jnp.int32),
      ],
  )
  return f(src, dst)


def _sc_layer(u0_flat, bls, bld, cnts, dinv2rep, ng, n, e):
  gc = ng // NC
  nw = NC * NS
  qpr = DH // L
  mesh = plsc.VectorSubcoreMesh(core_axis_name="c", subcore_axis_name="s")

  def body(u0, bls_h, bld_h, cnt_h, dv_h, s1, s2, s3, uw,
           wsrc0, wsrc1, wdst0, wdst1, cstage, rows0, rows1, acc, dvv,
           sl0, sl1, sg0, sg1):
    c = lax.axis_index("c")
    s = lax.axis_index("s")
    row0 = s * RPT
    pltpu.sync_copy(cnt_h, cstage)
    pltpu.sync_copy(dv_h.at[pl.ds(row0, RPT)], dvv)
    iot = lax.iota(jnp.int32, 16)
    cols = [iot + q * L for q in range(qpr)]
    zeros = jnp.zeros((L,), jnp.float32)

    def cnt_of(w):
      v = cstage[w]
      return jnp.sum(jnp.where(iot == s, v, 0))

    def lissue(w, wsrcb, wdstb, semb):
      pltpu.async_copy(bls_h.at[w].at[pl.ds(s * CAP, CAP)], wsrcb, semb)
      pltpu.async_copy(bld_h.at[w].at[pl.ds(s * CAP, CAP)], wdstb, semb)

    def lwait(w, wsrcb, wdstb, semb):
      pltpu.make_async_copy(bls_h.at[w].at[pl.ds(s * CAP, CAP)], wsrcb,
                            semb).wait()
      pltpu.make_async_copy(bld_h.at[w].at[pl.ds(s * CAP, CAP)], wdstb,
                            semb).wait()

    def accum(rowsb, wdstb, j):
      def ab(i, carry):
        for u in range(4):
          le = i * 4 + u
          dl = plsc.load_gather(wdstb, [jnp.full((L,), j * CH + le,
                                                 jnp.int32)])
          for q in range(qpr):
            plsc.addupdate_scatter(acc, [dl, cols[q]],
                                   rowsb[le, pl.ds(q * L, L)])
        return carry
      lax.fori_loop(0, CH // 4, ab, 0)

    def process_worker(usrc_g, wsrcb, wdstb, w):
      cnt = cnt_of(w)
      nchw = (cnt + (CH - 1)) // CH

      def gissue(j, rb, sg):
        pltpu.async_copy(usrc_g.at[wsrcb.at[pl.ds(j * CH, CH)]], rb, sg)

      def gwait(j, rb, sg):
        pltpu.make_async_copy(usrc_g.at[wsrcb.at[pl.ds(j * CH, CH)]], rb,
                              sg).wait()

      @pl.when(nchw > 0)
      def _():
        gissue(0, rows0, sg0)

        def qb(q, carry):
          j0 = 2 * q
          j1 = 2 * q + 1

          @pl.when(j1 < nchw)
          def _():
            gissue(j1, rows1, sg1)
          gwait(j0, rows0, sg0)
          accum(rows0, wdstb, j0)

          @pl.when(j1 < nchw)
          def _():
            @pl.when(j1 + 1 < nchw)
            def _():
              gissue(j1 + 1, rows0, sg0)
            gwait(j1, rows1, sg1)
            accum(rows1, wdstb, j1)
          return carry
        lax.fori_loop(0, (nchw + 1) // 2, qb, 0)

    for k in range(3):
      uin = u0 if k == 0 else uw
      sout = (s1, s2, s3)[k]
      for p in range(gc):
        g = c * gc + p
        ubase = g * n
        usrc_g = uin.at[pl.ds(ubase, n)]

        def zb(i, carry):
          acc[i // qpr, pl.ds((i % qpr) * L, L)] = zeros
          return carry
        lax.fori_loop(0, (RPT + 1) * qpr, zb, 0)

        lissue(0, wsrc0, wdst0, sl0)

        def wb(wp, carry, _usrc=usrc_g):
          w0 = 2 * wp
          w1 = 2 * wp + 1
          lissue(w1, wsrc1, wdst1, sl1)
          lwait(w0, wsrc0, wdst0, sl0)
          process_worker(_usrc, wsrc0, wdst0, w0)

          @pl.when(w1 + 1 < nw)
          def _():
            lissue(w1 + 1, wsrc0, wdst0, sl0)
          lwait(w1, wsrc1, wdst1, sl1)
          process_worker(_usrc, wsrc1, wdst1, w1)
          return carry
        lax.fori_loop(0, nw // 2, wb, 0)
        plsc.subcore_barrier()

        pltpu.sync_copy(acc.at[pl.ds(0, RPT)],
                        sout.at[pl.ds(ubase + row0, RPT)])
        if k < 2:
          def sb(rr, carry):
            dvrow = dvv[rr]
            for q in range(qpr):
              acc[rr, pl.ds(q * L, L)] = acc[rr, pl.ds(q * L, L)] * dvrow
            return carry
          lax.fori_loop(0, RPT, sb, 0)
          pltpu.sync_copy(acc.at[pl.ds(0, RPT)],
                          uw.at[pl.ds(ubase + row0, RPT)])
        plsc.subcore_barrier()

  f = pl.kernel(
      body,
      out_type=[jax.ShapeDtypeStruct((ng * n, DH), jnp.float32)] * 4,
      mesh=mesh,
      compiler_params=_SC_PARAMS,
      scratch_types=[
          pltpu.VMEM((CAP,), jnp.int32),
          pltpu.VMEM((CAP,), jnp.int32),
          pltpu.VMEM((CAP,), jnp.int32),
          pltpu.VMEM((CAP,), jnp.int32),
          pltpu.VMEM((NC * NS, 16), jnp.int32),
          pltpu.VMEM((CH, DH), jnp.float32),
          pltpu.VMEM((CH, DH), jnp.float32),
          pltpu.VMEM((RPT + 1, DH), jnp.float32),
          pltpu.VMEM((RPT, L), jnp.float32),
          pltpu.SemaphoreType.DMA,
          pltpu.SemaphoreType.DMA,
          pltpu.SemaphoreType.DMA,
          pltpu.SemaphoreType.DMA,
      ],
  )
  return f(u0_flat, bls, bld, cnts, dinv2rep)


def _dinv_block(deg_ref):
  deg = jnp.sum(deg_ref[0], axis=0)
  return jnp.where(deg > 0, lax.rsqrt(jnp.maximum(deg, 1e-12)), 0.0)


def _deg_spec():
  return pl.BlockSpec((1, NC * NS, _R), lambda i: (i, 0, 0))


def _split_groups(u, u_ref, ng):
  for g in range(ng):
    u_ref[g] = u[:, g * DH:(g + 1) * DH]


def _cat_groups(sref, ng):
  return jnp.concatenate([sref[g] for g in range(ng)], axis=1)


def _tc_prep(deg_p, x, W1, n, d_in, hid):
  g = n // _R
  ng = d_in // DH

  def body(deg_ref, x_ref, w_ref, dv_ref, u0_ref, acc_ref):
    dinv = _dinv_block(deg_ref)
    xb = x_ref[...]
    _split_groups(xb * dinv[:, None], u0_ref, ng)
    dv_ref[...] = jnp.broadcast_to((dinv * dinv)[:, None], (_R, L))
    acc_ref[...] = jnp.dot(xb, w_ref[0], preferred_element_type=jnp.float32)

  return pl.pallas_call(
      body,
      grid=(g,),
      in_specs=[
          _deg_spec(),
          pl.BlockSpec((_R, d_in), lambda i: (i, 0)),
          pl.BlockSpec(W1.shape, lambda i: (0, 0, 0)),
      ],
      out_specs=[
          pl.BlockSpec((_R, L), lambda i: (i, 0)),
          pl.BlockSpec((ng, _R, DH), lambda i: (0, i, 0)),
          pl.BlockSpec((_R, hid), lambda i: (i, 0)),
      ],
      out_shape=[
          jax.ShapeDtypeStruct((n, L), jnp.float32),
          jax.ShapeDtypeStruct((ng, n, DH), jnp.float32),
          jax.ShapeDtypeStruct((n, hid), jnp.float32),
      ],
  )(deg_p, x, W1)


def _tc_mid(deg_p, acc0, s1, s2, s3, W1, b1, a1, W2, n, d_in, hid):
  g = n // _R
  ng1 = d_in // DH
  ng2 = hid // DH

  def body(deg_ref, acc_ref, s1_ref, s2_ref, s3_ref, w1_ref, b1_ref, a1_ref,
           w2_ref, u0_ref, acc2_ref):
    dinv = _dinv_block(deg_ref)
    h = acc_ref[...]
    for k, sref in enumerate((s1_ref, s2_ref, s3_ref)):
      sk = _cat_groups(sref, ng1) * dinv[:, None]
      h = h + jnp.dot(sk, w1_ref[k + 1], preferred_element_type=jnp.float32)
    h = h + b1_ref[...]
    a = a1_ref[0, 0]
    h = jnp.where(h >= 0, h, a * h)
    _split_groups(h * dinv[:, None], u0_ref, ng2)
    acc2_ref[...] = jnp.dot(h, w2_ref[0], preferred_element_type=jnp.float32)

  sspec = pl.BlockSpec((ng1, _R, DH), lambda i: (0, i, 0))
  return pl.pallas_call(
      body,
      grid=(g,),
      in_specs=[
          _deg_spec(),
          pl.BlockSpec((_R, hid), lambda i: (i, 0)),
          sspec, sspec, sspec,
          pl.BlockSpec(W1.shape, lambda i: (0, 0, 0)),
          pl.BlockSpec((1, hid), lambda i: (0, 0)),
          pl.BlockSpec((1, 1), lambda i: (0, 0)),
          pl.BlockSpec(W2.shape, lambda i: (0, 0, 0)),
      ],
      out_specs=[
          pl.BlockSpec((ng2, _R, DH), lambda i: (0, i, 0)),
          pl.BlockSpec((_R, W2.shape[2]), lambda i: (i, 0)),
      ],
      out_shape=[
          jax.ShapeDtypeStruct((ng2, n, DH), jnp.float32),
          jax.ShapeDtypeStruct((n, W2.shape[2]), jnp.float32),
      ],
  )(deg_p, acc0, s1, s2, s3, W1, b1, a1, W2)


def _tc_final(deg_p, acc2, t1, t2, t3, W2, b2, a2, n, hid, d_out):
  g = n // _R
  ng = hid // DH

  def body(deg_ref, acc_ref, s1_ref, s2_ref, s3_ref, w_ref, b_ref, a_ref,
           y_ref):
    dinv = _dinv_block(deg_ref)
    h = acc_ref[...]
    for k, sref in enumerate((s1_ref, s2_ref, s3_ref)):
      sk = _cat_groups(sref, ng) * dinv[:, None]
      h = h + jnp.dot(sk, w_ref[k + 1], preferred_element_type=jnp.float32)
    h = h + b_ref[...]
    a = a_ref[0, 0]
    y_ref[...] = jnp.where(h >= 0, h, a * h)

  sspec = pl.BlockSpec((ng, _R, DH), lambda i: (0, i, 0))
  return pl.pallas_call(
      body,
      grid=(g,),
      in_specs=[
          _deg_spec(),
          pl.BlockSpec((_R, d_out), lambda i: (i, 0)),
          sspec, sspec, sspec,
          pl.BlockSpec(W2.shape, lambda i: (0, 0, 0)),
          pl.BlockSpec((1, d_out), lambda i: (0, 0)),
          pl.BlockSpec((1, 1), lambda i: (0, 0)),
      ],
      out_specs=pl.BlockSpec((_R, d_out), lambda i: (i, 0)),
      out_shape=jax.ShapeDtypeStruct((n, d_out), jnp.float32),
  )(deg_p, acc2, t1, t2, t3, W2, b2, a2)


def kernel(x, edge_index, W1, b1, a1, W2, b2, a2):
  n, d_in = x.shape
  e = edge_index.shape[1]
  hid = W1.shape[2]
  d_out = W2.shape[2]
  src = edge_index[0]
  dst = edge_index[1]
  b1r = b1.reshape(1, hid)
  a1r = a1.reshape(1, 1)
  b2r = b2.reshape(1, d_out)
  a2r = a2.reshape(1, 1)
  ng1 = d_in // DH
  ng2 = hid // DH

  deg_p, bls, bld, cnts = _sc_prep(src, dst, n, e)
  xp = x.reshape(RPT, NS, d_in).transpose(1, 0, 2).reshape(n, d_in)
  dinv2rep, u0, acc0 = _tc_prep(deg_p, xp, W1, n, d_in, hid)
  s1, s2, s3, _ = _sc_layer(u0.reshape(ng1 * n, DH), bls, bld, cnts,
                            dinv2rep, ng1, n, e)
  rs = lambda v, ng: v.reshape(ng, n, DH)
  u0b, acc2 = _tc_mid(deg_p, acc0, rs(s1, ng1), rs(s2, ng1), rs(s3, ng1),
                      W1, b1r, a1r, W2, n, d_in, hid)
  t1, t2, t3, _ = _sc_layer(u0b.reshape(ng2 * n, DH), bls, bld, cnts,
                            dinv2rep, ng2, n, e)
  yp = _tc_final(deg_p, acc2, rs(t1, ng2), rs(t2, ng2), rs(t3, ng2),
                 W2, b2r, a2r, n, hid, d_out)
  return yp.reshape(NS, RPT, d_out).transpose(1, 0, 2).reshape(n, d_out)


# speedup vs baseline: 3.1629x; 3.1629x over previous
import jax
import jax.numpy as jnp
from jax import lax
from jax.experimental import pallas as pl
from jax.experimental.pallas import tpu as pltpu
from jax.experimental.pallas import tpu_sc as plsc

NC = 2
NS = 16
L = 16
CH = 80
DH = 64
ZR = 125
_R = 1000

_SC_PARAMS = pltpu.CompilerParams(
    needs_layout_passes=False, use_tc_tiling_on_sc=False)


def _sc_degree(dst, n, e):
  nw = NC * NS
  epw = e // nw
  nb = n // _R
  mesh = plsc.VectorSubcoreMesh(core_axis_name="c", subcore_axis_name="s")

  def body(dst_hbm, out_hbm, didx, deg):
    c = lax.axis_index("c")
    s = lax.axis_index("s")
    wid = s * NC + c
    pltpu.sync_copy(dst_hbm.at[pl.ds(wid * epw, epw)], didx)
    zeros = jnp.zeros((L,), jnp.float32)

    def zb(i, carry):
      deg[pl.ds(i * L, L)] = zeros
      return carry
    lax.fori_loop(0, n // L, zb, 0)

    ones = jnp.ones((L,), jnp.float32)

    def hb(i, carry):
      idx = didx[pl.ds(i * L, L)]
      plsc.addupdate_scatter(deg, [idx], ones)
      return carry
    lax.fori_loop(0, epw // L, hb, 0)
    for b in range(nb):
      pltpu.sync_copy(deg.at[pl.ds(b * _R, _R)], out_hbm.at[b].at[wid])

  f = pl.kernel(
      body,
      out_type=jax.ShapeDtypeStruct((nb, nw, _R), jnp.float32),
      mesh=mesh,
      compiler_params=_SC_PARAMS,
      scratch_types=[
          pltpu.VMEM((epw,), jnp.int32),
          pltpu.VMEM((n,), jnp.float32),
      ],
  )
  return f(dst)


def _sc_layer(u0_flat, src, dst, dinv2rep, ng, n, e):
  gc = ng // NC
  eps = e // NS
  nch = eps // CH
  rpt = n // NS
  mesh = plsc.VectorSubcoreMesh(core_axis_name="c", subcore_axis_name="s")

  def body(u0, src_h, dst_h, dv_h, s1, s2, s3, uw,
           sidx, didxall, rows0, rows1, zbuf, scbuf, dvv, acc,
           sem0, sem1):
    c = lax.axis_index("c")
    s = lax.axis_index("s")
    ebase = s * eps
    row0 = s * rpt
    pltpu.sync_copy(src_h.at[pl.ds(ebase, eps)], sidx)
    pltpu.sync_copy(dst_h.at[pl.ds(s * nch, nch)], didxall)
    pltpu.sync_copy(dv_h.at[pl.ds(row0, rpt)], dvv)

    off = jnp.full((L,), c * gc * n, jnp.int32)

    def ob(i, carry):
      sidx[pl.ds(i * L, L)] = sidx[pl.ds(i * L, L)] + off
      return carry
    lax.fori_loop(0, eps // L, ob, 0)

    zeros = jnp.zeros((L,), jnp.float32)
    qpr = DH // L

    def zb(i, carry):
      zbuf[i // qpr, pl.ds((i % qpr) * L, L)] = zeros
      return carry
    lax.fori_loop(0, ZR * qpr, zb, 0)

    bump = jnp.full((L,), n, jnp.int32)

    def bumpidx(i, carry):
      sidx[pl.ds(i * L, L)] = sidx[pl.ds(i * L, L)] + bump
      return carry

    def gissue(uin, j, rbuf, sem):
      pltpu.async_copy(uin.at[sidx.at[pl.ds(j * CH, CH)]], rbuf, sem)

    def gwait(uin, j, rbuf, sem):
      pltpu.make_async_copy(uin.at[sidx.at[pl.ds(j * CH, CH)]], rbuf,
                            sem).wait()

    def scat(j, rbuf):
      pltpu.sync_copy(rbuf, acc.at[didxall.at[j]], add=True)

    for k in range(3):
      uin = u0 if k == 0 else uw
      sout = (s1, s2, s3)[k]
      for p in range(gc):
        if p > 0:
          lax.fori_loop(0, eps // L, bumpidx, 0)
        g = c * gc + p
        for r in range(rpt // ZR):
          pltpu.sync_copy(zbuf, acc.at[pl.ds(row0 + r * ZR, ZR)])
        plsc.subcore_barrier()

        gissue(uin, 0, rows0, sem0)

        def eb(jj, carry, _uin=uin):
          j0 = 2 * jj
          j1 = 2 * jj + 1
          gissue(_uin, j1, rows1, sem1)
          gwait(_uin, j0, rows0, sem0)
          scat(j0, rows0)

          @pl.when(j1 + 1 < nch)
          def _():
            gissue(_uin, j1 + 1, rows0, sem0)
          gwait(_uin, j1, rows1, sem1)
          scat(j1, rows1)
          return carry
        lax.fori_loop(0, nch // 2, eb, 0)
        plsc.subcore_barrier()

        for r in range(rpt // ZR):
          rr0 = row0 + r * ZR
          pltpu.sync_copy(acc.at[pl.ds(rr0, ZR)],
                          sout.at[pl.ds(g * n + rr0, ZR)])
        if k < 2:
          for r in range(rpt // ZR):
            rr0 = row0 + r * ZR
            pltpu.sync_copy(acc.at[pl.ds(rr0, ZR)], scbuf)

            def sb(q, carry, _r=r):
              rr = q // qpr
              qq = q % qpr
              dvrow = dvv[_r * ZR + rr]
              scbuf[rr, pl.ds(qq * L, L)] = (
                  scbuf[rr, pl.ds(qq * L, L)] * dvrow)
              return carry
            lax.fori_loop(0, ZR * qpr, sb, 0)
            pltpu.sync_copy(scbuf, uw.at[pl.ds(g * n + rr0, ZR)])
        plsc.subcore_barrier()
      if gc > 1:
        off2 = jnp.full((L,), (gc - 1) * n, jnp.int32)

        def rewind(i, carry):
          sidx[pl.ds(i * L, L)] = sidx[pl.ds(i * L, L)] - off2
          return carry
        lax.fori_loop(0, eps // L, rewind, 0)

  f = pl.kernel(
      body,
      out_type=[jax.ShapeDtypeStruct((ng * n, DH), jnp.float32)] * 4,
      mesh=mesh,
      compiler_params=_SC_PARAMS,
      scratch_types=[
          pltpu.VMEM((eps,), jnp.int32),
          pltpu.VMEM((nch, CH), jnp.int32),
          pltpu.VMEM((CH, DH), jnp.float32),
          pltpu.VMEM((CH, DH), jnp.float32),
          pltpu.VMEM((ZR, DH), jnp.float32),
          pltpu.VMEM((ZR, DH), jnp.float32),
          pltpu.VMEM((rpt, L), jnp.float32),
          pltpu.VMEM_SHARED((n, DH), jnp.float32),
          pltpu.SemaphoreType.DMA,
          pltpu.SemaphoreType.DMA,
      ],
  )
  return f(u0_flat, src, dst, dinv2rep)


def _dinv_block(deg_ref):
  deg = jnp.sum(deg_ref[0], axis=0)
  return jnp.where(deg > 0, lax.rsqrt(jnp.maximum(deg, 1e-12)), 0.0)


def _deg_spec():
  return pl.BlockSpec((1, NC * NS, _R), lambda i: (i, 0, 0))


def _split_groups(u, u_ref, ng):
  for g in range(ng):
    u_ref[g] = u[:, g * DH:(g + 1) * DH]


def _cat_groups(sref, ng):
  return jnp.concatenate([sref[g] for g in range(ng)], axis=1)


def _tc_prep(deg_p, x, W1, n, d_in, hid):
  g = n // _R
  ng = d_in // DH

  def body(deg_ref, x_ref, w_ref, dv_ref, u0_ref, acc_ref):
    dinv = _dinv_block(deg_ref)
    xb = x_ref[...]
    _split_groups(xb * dinv[:, None], u0_ref, ng)
    dv_ref[...] = jnp.broadcast_to((dinv * dinv)[:, None], (_R, L))
    acc_ref[...] = jnp.dot(xb, w_ref[0], preferred_element_type=jnp.float32)

  return pl.pallas_call(
      body,
      grid=(g,),
      in_specs=[
          _deg_spec(),
          pl.BlockSpec((_R, d_in), lambda i: (i, 0)),
          pl.BlockSpec(W1.shape, lambda i: (0, 0, 0)),
      ],
      out_specs=[
          pl.BlockSpec((_R, L), lambda i: (i, 0)),
          pl.BlockSpec((ng, _R, DH), lambda i: (0, i, 0)),
          pl.BlockSpec((_R, hid), lambda i: (i, 0)),
      ],
      out_shape=[
          jax.ShapeDtypeStruct((n, L), jnp.float32),
          jax.ShapeDtypeStruct((ng, n, DH), jnp.float32),
          jax.ShapeDtypeStruct((n, hid), jnp.float32),
      ],
  )(deg_p, x, W1)


def _tc_mid(deg_p, acc0, s1, s2, s3, W1, b1, a1, W2, n, d_in, hid):
  g = n // _R
  ng1 = d_in // DH
  ng2 = hid // DH

  def body(deg_ref, acc_ref, s1_ref, s2_ref, s3_ref, w1_ref, b1_ref, a1_ref,
           w2_ref, u0_ref, acc2_ref):
    dinv = _dinv_block(deg_ref)
    h = acc_ref[...]
    for k, sref in enumerate((s1_ref, s2_ref, s3_ref)):
      sk = _cat_groups(sref, ng1) * dinv[:, None]
      h = h + jnp.dot(sk, w1_ref[k + 1], preferred_element_type=jnp.float32)
    h = h + b1_ref[...]
    a = a1_ref[0, 0]
    h = jnp.where(h >= 0, h, a * h)
    _split_groups(h * dinv[:, None], u0_ref, ng2)
    acc2_ref[...] = jnp.dot(h, w2_ref[0], preferred_element_type=jnp.float32)

  sspec = pl.BlockSpec((ng1, _R, DH), lambda i: (0, i, 0))
  return pl.pallas_call(
      body,
      grid=(g,),
      in_specs=[
          _deg_spec(),
          pl.BlockSpec((_R, hid), lambda i: (i, 0)),
          sspec, sspec, sspec,
          pl.BlockSpec(W1.shape, lambda i: (0, 0, 0)),
          pl.BlockSpec((1, hid), lambda i: (0, 0)),
          pl.BlockSpec((1, 1), lambda i: (0, 0)),
          pl.BlockSpec(W2.shape, lambda i: (0, 0, 0)),
      ],
      out_specs=[
          pl.BlockSpec((ng2, _R, DH), lambda i: (0, i, 0)),
          pl.BlockSpec((_R, W2.shape[2]), lambda i: (i, 0)),
      ],
      out_shape=[
          jax.ShapeDtypeStruct((ng2, n, DH), jnp.float32),
          jax.ShapeDtypeStruct((n, W2.shape[2]), jnp.float32),
      ],
  )(deg_p, acc0, s1, s2, s3, W1, b1, a1, W2)


def _tc_final(deg_p, acc2, t1, t2, t3, W2, b2, a2, n, hid, d_out):
  g = n // _R
  ng = hid // DH

  def body(deg_ref, acc_ref, s1_ref, s2_ref, s3_ref, w_ref, b_ref, a_ref,
           y_ref):
    dinv = _dinv_block(deg_ref)
    h = acc_ref[...]
    for k, sref in enumerate((s1_ref, s2_ref, s3_ref)):
      sk = _cat_groups(sref, ng) * dinv[:, None]
      h = h + jnp.dot(sk, w_ref[k + 1], preferred_element_type=jnp.float32)
    h = h + b_ref[...]
    a = a_ref[0, 0]
    y_ref[...] = jnp.where(h >= 0, h, a * h)

  sspec = pl.BlockSpec((ng, _R, DH), lambda i: (0, i, 0))
  return pl.pallas_call(
      body,
      grid=(g,),
      in_specs=[
          _deg_spec(),
          pl.BlockSpec((_R, d_out), lambda i: (i, 0)),
          sspec, sspec, sspec,
          pl.BlockSpec(W2.shape, lambda i: (0, 0, 0)),
          pl.BlockSpec((1, d_out), lambda i: (0, 0)),
          pl.BlockSpec((1, 1), lambda i: (0, 0)),
      ],
      out_specs=pl.BlockSpec((_R, d_out), lambda i: (i, 0)),
      out_shape=jax.ShapeDtypeStruct((n, d_out), jnp.float32),
  )(deg_p, acc2, t1, t2, t3, W2, b2, a2)


def kernel(x, edge_index, W1, b1, a1, W2, b2, a2):
  n, d_in = x.shape
  e = edge_index.shape[1]
  hid = W1.shape[2]
  d_out = W2.shape[2]
  src = edge_index[0]
  dst = edge_index[1]
  b1r = b1.reshape(1, hid)
  a1r = a1.reshape(1, 1)
  b2r = b2.reshape(1, d_out)
  a2r = a2.reshape(1, 1)
  ng1 = d_in // DH
  ng2 = hid // DH

  deg_p = _sc_degree(dst, n, e)
  dinv2rep, u0, acc0 = _tc_prep(deg_p, x, W1, n, d_in, hid)
  dst2d = dst.reshape(e // CH, CH)
  s1, s2, s3, _ = _sc_layer(u0.reshape(ng1 * n, DH), src, dst2d, dinv2rep,
                            ng1, n, e)
  rs1 = lambda v, ng: v.reshape(ng, n, DH)
  u0b, acc2 = _tc_mid(deg_p, acc0, rs1(s1, ng1), rs1(s2, ng1), rs1(s3, ng1),
                      W1, b1r, a1r, W2, n, d_in, hid)
  t1, t2, t3, _ = _sc_layer(u0b.reshape(ng2 * n, DH), src, dst2d, dinv2rep,
                            ng2, n, e)
  return _tc_final(deg_p, acc2, rs1(t1, ng2), rs1(t2, ng2), rs1(t3, ng2),
                   W2, b2r, a2r, n, hid, d_out)


# speedup vs baseline: 4.6252x; 1.4623x over previous
import jax
import jax.numpy as jnp
from jax import lax
from jax.experimental import pallas as pl
from jax.experimental.pallas import tpu as pltpu
from jax.experimental.pallas import tpu_sc as plsc

NC = 2
NS = 16
L = 16
CH = 80
DH = 64
ZR = 125
_R = 1000

_SC_PARAMS = pltpu.CompilerParams(
    needs_layout_passes=False, use_tc_tiling_on_sc=False)


def _sc_degree(dst, n, e):
  nw = NC * NS
  epw = e // nw
  nb = n // _R
  mesh = plsc.VectorSubcoreMesh(core_axis_name="c", subcore_axis_name="s")

  def body(dst_hbm, out_hbm, didx, deg):
    c = lax.axis_index("c")
    s = lax.axis_index("s")
    wid = s * NC + c
    pltpu.sync_copy(dst_hbm.at[pl.ds(wid * epw, epw)], didx)
    zeros = jnp.zeros((L,), jnp.float32)

    def zb(i, carry):
      deg[pl.ds(i * L, L)] = zeros
      return carry
    lax.fori_loop(0, n // L, zb, 0)

    ones = jnp.ones((L,), jnp.float32)

    def hb(i, carry):
      idx = didx[pl.ds(i * L, L)]
      plsc.addupdate_scatter(deg, [idx], ones)
      return carry
    lax.fori_loop(0, epw // L, hb, 0)
    for b in range(nb):
      pltpu.sync_copy(deg.at[pl.ds(b * _R, _R)], out_hbm.at[b].at[wid])

  f = pl.kernel(
      body,
      out_type=jax.ShapeDtypeStruct((nb, nw, _R), jnp.float32),
      mesh=mesh,
      compiler_params=_SC_PARAMS,
      scratch_types=[
          pltpu.VMEM((epw,), jnp.int32),
          pltpu.VMEM((n,), jnp.float32),
      ],
  )
  return f(dst)


def _sc_layer(u0_flat, src, dst, dinv2rep, ng, n, e, vps=None):
  horner = vps is not None
  gc = ng // NC
  eps = e // NS
  nch = eps // CH
  rpt = n // NS
  mesh = plsc.VectorSubcoreMesh(core_axis_name="c", subcore_axis_name="s")

  def body(*refs):
    if horner:
      (u0, src_h, dst_h, dv_h, w1_h, w2_h, sfin, uw,
       sidx, didxall, rows0, rows1, zbuf, scbuf, vbuf, dvv, acc,
       sem0, sem1) = refs
      souts = (None, None, sfin)
      vadds = (w1_h, w2_h, None)
    else:
      (u0, src_h, dst_h, dv_h, s1, s2, s3, uw,
       sidx, didxall, rows0, rows1, zbuf, scbuf, vbuf, dvv, acc,
       sem0, sem1) = refs
      souts = (s1, s2, s3)
      vadds = (None, None, None)
    c = lax.axis_index("c")
    s = lax.axis_index("s")
    ebase = s * eps
    row0 = s * rpt
    pltpu.sync_copy(src_h.at[pl.ds(ebase, eps)], sidx)
    pltpu.sync_copy(dst_h.at[pl.ds(s * nch, nch)], didxall)
    pltpu.sync_copy(dv_h.at[pl.ds(row0, rpt)], dvv)

    off = jnp.full((L,), c * gc * n, jnp.int32)

    def ob(i, carry):
      sidx[pl.ds(i * L, L)] = sidx[pl.ds(i * L, L)] + off
      return carry
    lax.fori_loop(0, eps // L, ob, 0)

    zeros = jnp.zeros((L,), jnp.float32)
    qpr = DH // L

    def zb(i, carry):
      zbuf[i // qpr, pl.ds((i % qpr) * L, L)] = zeros
      return carry
    lax.fori_loop(0, ZR * qpr, zb, 0)

    bump = jnp.full((L,), n, jnp.int32)

    def bumpidx(i, carry):
      sidx[pl.ds(i * L, L)] = sidx[pl.ds(i * L, L)] + bump
      return carry

    def gissue(uin, j, rbuf, sem):
      pltpu.async_copy(uin.at[sidx.at[pl.ds(j * CH, CH)]], rbuf, sem)

    def gwait(uin, j, rbuf, sem):
      pltpu.make_async_copy(uin.at[sidx.at[pl.ds(j * CH, CH)]], rbuf,
                            sem).wait()

    def scat(j, rbuf):
      pltpu.sync_copy(rbuf, acc.at[didxall.at[j]], add=True)

    for k in range(3):
      uin = u0 if k == 0 else uw
      sout = souts[k]
      vadd = vadds[k]
      for p in range(gc):
        if p > 0:
          lax.fori_loop(0, eps // L, bumpidx, 0)
        g = c * gc + p
        for r in range(rpt // ZR):
          pltpu.sync_copy(zbuf, acc.at[pl.ds(row0 + r * ZR, ZR)])
        plsc.subcore_barrier()

        gissue(uin, 0, rows0, sem0)

        def eb(jj, carry, _uin=uin):
          j0 = 2 * jj
          j1 = 2 * jj + 1
          gissue(_uin, j1, rows1, sem1)
          gwait(_uin, j0, rows0, sem0)
          scat(j0, rows0)

          @pl.when(j1 + 1 < nch)
          def _():
            gissue(_uin, j1 + 1, rows0, sem0)
          gwait(_uin, j1, rows1, sem1)
          scat(j1, rows1)
          return carry
        lax.fori_loop(0, nch // 2, eb, 0)
        plsc.subcore_barrier()

        if sout is not None:
          for r in range(rpt // ZR):
            rr0 = row0 + r * ZR
            pltpu.sync_copy(acc.at[pl.ds(rr0, ZR)],
                            sout.at[pl.ds(g * n + rr0, ZR)])
        if k < 2:
          for r in range(rpt // ZR):
            rr0 = row0 + r * ZR
            pltpu.sync_copy(acc.at[pl.ds(rr0, ZR)], scbuf)
            if vadd is not None:
              pltpu.sync_copy(vadd.at[pl.ds(g * n + rr0, ZR)], vbuf)

            def sb(q, carry, _r=r, _va=vadd is not None):
              rr = q // qpr
              qq = q % qpr
              dvrow = dvv[_r * ZR + rr]
              v = scbuf[rr, pl.ds(qq * L, L)] * dvrow
              if _va:
                v = v + vbuf[rr, pl.ds(qq * L, L)]
              scbuf[rr, pl.ds(qq * L, L)] = v
              return carry
            lax.fori_loop(0, ZR * qpr, sb, 0)
            pltpu.sync_copy(scbuf, uw.at[pl.ds(g * n + rr0, ZR)])
        plsc.subcore_barrier()
      if gc > 1:
        off2 = jnp.full((L,), (gc - 1) * n, jnp.int32)

        def rewind(i, carry):
          sidx[pl.ds(i * L, L)] = sidx[pl.ds(i * L, L)] - off2
          return carry
        lax.fori_loop(0, eps // L, rewind, 0)

  n_out = 2 if horner else 4
  f = pl.kernel(
      body,
      out_type=[jax.ShapeDtypeStruct((ng * n, DH), jnp.float32)] * n_out,
      mesh=mesh,
      compiler_params=_SC_PARAMS,
      scratch_types=[
          pltpu.VMEM((eps,), jnp.int32),
          pltpu.VMEM((nch, CH), jnp.int32),
          pltpu.VMEM((CH, DH), jnp.float32),
          pltpu.VMEM((CH, DH), jnp.float32),
          pltpu.VMEM((ZR, DH), jnp.float32),
          pltpu.VMEM((ZR, DH), jnp.float32),
          pltpu.VMEM((ZR, DH), jnp.float32),
          pltpu.VMEM((rpt, L), jnp.float32),
          pltpu.VMEM_SHARED((n, DH), jnp.float32),
          pltpu.SemaphoreType.DMA,
          pltpu.SemaphoreType.DMA,
      ],
  )
  if horner:
    return f(u0_flat, src, dst, dinv2rep, vps[0], vps[1])
  return f(u0_flat, src, dst, dinv2rep)


def _dinv_block(deg_ref):
  deg = jnp.sum(deg_ref[0], axis=0)
  return jnp.where(deg > 0, lax.rsqrt(jnp.maximum(deg, 1e-12)), 0.0)


def _deg_spec():
  return pl.BlockSpec((1, NC * NS, _R), lambda i: (i, 0, 0))


def _split_groups(u, u_ref, ng, dh):
  for g in range(ng):
    u_ref[g] = u[:, g * dh:(g + 1) * dh]


def _cat_groups(sref, ng):
  return jnp.concatenate([sref[g] for g in range(ng)], axis=1)


def _tc_prep(deg_p, x, W1, n, d_in, hid):
  g = n // _R
  ng = d_in // DH

  def body(deg_ref, x_ref, w_ref, dv_ref, u0_ref, acc_ref):
    dinv = _dinv_block(deg_ref)
    xb = x_ref[...]
    _split_groups(xb * dinv[:, None], u0_ref, ng, DH)
    dv_ref[...] = jnp.broadcast_to((dinv * dinv)[:, None], (_R, L))
    acc_ref[...] = jnp.dot(xb, w_ref[0], preferred_element_type=jnp.float32)

  return pl.pallas_call(
      body,
      grid=(g,),
      in_specs=[
          _deg_spec(),
          pl.BlockSpec((_R, d_in), lambda i: (i, 0)),
          pl.BlockSpec(W1.shape, lambda i: (0, 0, 0)),
      ],
      out_specs=[
          pl.BlockSpec((_R, L), lambda i: (i, 0)),
          pl.BlockSpec((ng, _R, DH), lambda i: (0, i, 0)),
          pl.BlockSpec((_R, hid), lambda i: (i, 0)),
      ],
      out_shape=[
          jax.ShapeDtypeStruct((n, L), jnp.float32),
          jax.ShapeDtypeStruct((ng, n, DH), jnp.float32),
          jax.ShapeDtypeStruct((n, hid), jnp.float32),
      ],
  )(deg_p, x, W1)


def _tc_mid(deg_p, acc0, s1, s2, s3, W1, b1, a1, W2, n, d_in, hid):
  g = n // _R
  ng1 = d_in // DH
  d_out = W2.shape[2]
  ngo = d_out // DH

  def body(deg_ref, acc_ref, s1_ref, s2_ref, s3_ref, w1_ref, b1_ref, a1_ref,
           w2_ref, u0_ref, h1_ref, h2_ref, acc2_ref):
    dinv = _dinv_block(deg_ref)
    h = acc_ref[...]
    for k, sref in enumerate((s1_ref, s2_ref, s3_ref)):
      sk = _cat_groups(sref, ng1) * dinv[:, None]
      h = h + jnp.dot(sk, w1_ref[k + 1], preferred_element_type=jnp.float32)
    h = h + b1_ref[...]
    a = a1_ref[0, 0]
    h = jnp.where(h >= 0, h, a * h)
    vd = dinv[:, None]
    _split_groups(
        jnp.dot(h, w2_ref[3], preferred_element_type=jnp.float32) * vd,
        u0_ref, ngo, DH)
    _split_groups(
        jnp.dot(h, w2_ref[2], preferred_element_type=jnp.float32) * vd,
        h1_ref, ngo, DH)
    _split_groups(
        jnp.dot(h, w2_ref[1], preferred_element_type=jnp.float32) * vd,
        h2_ref, ngo, DH)
    acc2_ref[...] = jnp.dot(h, w2_ref[0], preferred_element_type=jnp.float32)

  sspec = pl.BlockSpec((ng1, _R, DH), lambda i: (0, i, 0))
  ospec = pl.BlockSpec((ngo, _R, DH), lambda i: (0, i, 0))
  oshape = jax.ShapeDtypeStruct((ngo, n, DH), jnp.float32)
  return pl.pallas_call(
      body,
      grid=(g,),
      in_specs=[
          _deg_spec(),
          pl.BlockSpec((_R, hid), lambda i: (i, 0)),
          sspec, sspec, sspec,
          pl.BlockSpec(W1.shape, lambda i: (0, 0, 0)),
          pl.BlockSpec((1, hid), lambda i: (0, 0)),
          pl.BlockSpec((1, 1), lambda i: (0, 0)),
          pl.BlockSpec(W2.shape, lambda i: (0, 0, 0)),
      ],
      out_specs=[
          ospec, ospec, ospec,
          pl.BlockSpec((_R, d_out), lambda i: (i, 0)),
      ],
      out_shape=[
          oshape, oshape, oshape,
          jax.ShapeDtypeStruct((n, d_out), jnp.float32),
      ],
  )(deg_p, acc0, s1, s2, s3, W1, b1, a1, W2)


def _tc_final(deg_p, acc2, t3, b2, a2, n, d_out):
  g = n // _R
  ng = d_out // DH

  def body(deg_ref, acc_ref, s_ref, b_ref, a_ref, y_ref):
    dinv = _dinv_block(deg_ref)
    h = acc_ref[...] + _cat_groups(s_ref, ng) * dinv[:, None] + b_ref[...]
    a = a_ref[0, 0]
    y_ref[...] = jnp.where(h >= 0, h, a * h)

  return pl.pallas_call(
      body,
      grid=(g,),
      in_specs=[
          _deg_spec(),
          pl.BlockSpec((_R, d_out), lambda i: (i, 0)),
          pl.BlockSpec((ng, _R, DH), lambda i: (0, i, 0)),
          pl.BlockSpec((1, d_out), lambda i: (0, 0)),
          pl.BlockSpec((1, 1), lambda i: (0, 0)),
      ],
      out_specs=pl.BlockSpec((_R, d_out), lambda i: (i, 0)),
      out_shape=jax.ShapeDtypeStruct((n, d_out), jnp.float32),
  )(deg_p, acc2, t3, b2, a2)


def kernel(x, edge_index, W1, b1, a1, W2, b2, a2):
  n, d_in = x.shape
  e = edge_index.shape[1]
  hid = W1.shape[2]
  d_out = W2.shape[2]
  src = edge_index[0]
  dst = edge_index[1]
  b1r = b1.reshape(1, hid)
  a1r = a1.reshape(1, 1)
  b2r = b2.reshape(1, d_out)
  a2r = a2.reshape(1, 1)
  ng1 = d_in // DH
  ngo = d_out // DH

  deg_p = _sc_degree(dst, n, e)
  dinv2rep, u0, acc0 = _tc_prep(deg_p, x, W1, n, d_in, hid)
  dst2d = dst.reshape(e // CH, CH)
  s1, s2, s3, _ = _sc_layer(u0.reshape(ng1 * n, DH), src, dst2d, dinv2rep,
                            ng1, n, e)
  rs = lambda v, ng: v.reshape(ng, n, DH)
  u0b, w1, w2, acc2 = _tc_mid(deg_p, acc0, rs(s1, ng1), rs(s2, ng1),
                              rs(s3, ng1), W1, b1r, a1r, W2, n, d_in, hid)
  t3, _ = _sc_layer(u0b.reshape(ngo * n, DH), src, dst2d, dinv2rep,
                    ngo, n, e,
                    vps=(w1.reshape(ngo * n, DH), w2.reshape(ngo * n, DH)))
  return _tc_final(deg_p, acc2, rs(t3, ngo), b2r, a2r, n, d_out)
